# baseline (speedup 1.0000x reference)
import functools

import jax
import jax.numpy as jnp
from jax import lax
from jax.experimental import pallas as pl
from jax.experimental.pallas import tpu as pltpu
from jax.experimental.pallas import tpu_sc as plsc

N_ROWS = 1_000_000
NW = 32
ROWS_W = 31_248
TAIL_ROW0 = NW * ROWS_W
TAIL_ROWS = N_ROWS - TAIL_ROW0
CHUNK = 4_000
NFULL = 7
LAST = ROWS_W - NFULL * CHUNK
EPS = 1e-7

_mesh = plsc.VectorSubcoreMesh(core_axis_name="c", subcore_axis_name="s")


def _sc_body(tgt_hbm, reg_hbm, lab_hbm, out_hbm, tgt_v, reg_v, lab_v, part_v):
    wid = lax.axis_index("s") * 2 + lax.axis_index("c")
    base_row = pl.multiple_of(wid * ROWS_W, 8)
    q = jnp.arange(16, dtype=jnp.int32) >> 2

    zero = jnp.zeros((16,), jnp.float32)
    acc = (zero, zero, zero)

    def chunk_accumulate(row0, nrows, acc, scale):
        pltpu.sync_copy(tgt_hbm.at[pl.ds(row0 * 4, nrows * 4)],
                        tgt_v.at[pl.ds(0, nrows * 4)])
        pltpu.sync_copy(reg_hbm.at[pl.ds(row0 * 4, nrows * 4)],
                        reg_v.at[pl.ds(0, nrows * 4)])
        pltpu.sync_copy(lab_hbm.at[pl.ds(row0, nrows)],
                        lab_v.at[pl.ds(0, nrows)])

        def it(u, acc):
            aa, av, ap = acc
            lab16 = lab_v[pl.ds(u * 16, 16)]
            if scale is None:
                one = 1.0
            else:
                one = scale
            av = av + jnp.where(lab16 != -1, one, 0.0)
            ap = ap + jnp.where(lab16 == 1, one, 0.0)
            for j in range(4):
                e = u * 64 + j * 16
                t = tgt_v[pl.ds(e, 16)]
                r = reg_v[pl.ds(e, 16)]
                x = t - r
                ax = jnp.abs(x)
                m = jnp.minimum(ax, 1.0)
                y = m * (ax - 0.5 * m)
                labg = plsc.load_gather(lab_v, [u * 16 + j * 4 + q])
                if scale is not None:
                    y = y * scale
                aa = aa + jnp.where(labg == 1, y, 0.0)
            return aa, av, ap

        return lax.fori_loop(0, nrows // 16, it, acc)

    for i in range(NFULL):
        acc = chunk_accumulate(base_row + i * CHUNK, CHUNK, acc, None)
    acc = chunk_accumulate(base_row + NFULL * CHUNK, LAST, acc, None)
    tail_scale = jnp.where(wid == NW - 1, 1.0, 0.0)
    acc = chunk_accumulate(TAIL_ROW0, TAIL_ROWS, acc, tail_scale)

    part_v[0, :] = acc[0]
    part_v[1, :] = acc[1]
    part_v[2, :] = acc[2]
    pltpu.sync_copy(part_v, out_hbm.at[wid])


_sc_partials = pl.kernel(
    _sc_body,
    out_type=jax.ShapeDtypeStruct((NW, 3, 16), jnp.float32),
    mesh=_mesh,
    compiler_params=pltpu.CompilerParams(needs_layout_passes=False),
    scratch_types=[
        pltpu.VMEM((CHUNK * 4,), jnp.float32),
        pltpu.VMEM((CHUNK * 4,), jnp.float32),
        pltpu.VMEM((CHUNK,), jnp.int32),
        pltpu.VMEM((3, 16), jnp.float32),
    ],
)


def _combine_body(p_ref, o_ref):
    p = p_ref[...]
    a = jnp.sum(p[:, 0, :])
    nv = jnp.sum(p[:, 1, :])
    npos = jnp.sum(p[:, 2, :])
    o_ref[0, 0] = a / (EPS * nv + npos)


_combine = pl.pallas_call(
    _combine_body,
    out_shape=jax.ShapeDtypeStruct((1, 1), jnp.float32),
    out_specs=pl.BlockSpec(memory_space=pltpu.SMEM),
)


@jax.jit
def kernel(rpn_bbox_targets, rpn_regression, rpn_labels):
    tgt = rpn_bbox_targets.reshape(-1)
    reg = rpn_regression.reshape(-1)
    lab = rpn_labels.astype(jnp.int32)
    partials = _sc_partials(tgt, reg, lab)
    loss = _combine(partials)[0, 0]
    return rpn_regression, loss
